# baseline (speedup 1.0000x reference)
import jax
import jax.numpy as jnp
from jax import lax
from jax.experimental import pallas as pl
from jax.experimental.pallas import tpu as pltpu
from jax.experimental.pallas import tpu_sc as plsc

NNZ = 819200
NUM_IDS = 16384
EMBED_DIM = 64
NUM_CORES = 2
NUM_SUBCORES = 16
NUM_WORKERS = NUM_CORES * NUM_SUBCORES
ROWS_PER_TILE = NUM_IDS // NUM_WORKERS
ACCW = ROWS_PER_TILE * EMBED_DIM
K = 128
NTAB = 500000
PAD = 1024
NBI = 4


def _sc_body(row_hbm, colp_hbm, w0_hbm, w1_hbm, glo_hbm, ghi_hbm,
             emb_hbm, out_hbm,
             gb_v,
             r0, r1, r2, r3, c0, c1, c2, c3,
             wa0, wa1, wa2, wa3, wb0, wb1, wb2, wb3,
             rows0, rows1, acc,
             si0, si1, si2, si3, sg0, sg1):
    rvs = (r0, r1, r2, r3)
    cvs = (c0, c1, c2, c3)
    w0s = (wa0, wa1, wa2, wa3)
    w1s = (wb0, wb1, wb2, wb3)
    rows = (rows0, rows1)
    sem_i = (si0, si1, si2, si3)
    sem_g = (sg0, sg1)

    c = lax.axis_index("c")
    s = lax.axis_index("s")
    w = c * NUM_SUBCORES + s

    zv = jnp.zeros((16,), jnp.float32)

    def zacc(j, carry):
        acc[pl.ds(j * 16, 16)] = zv
        return carry

    lax.fori_loop(0, ACCW // 16, zacc, 0)

    pltpu.sync_copy(glo_hbm, gb_v.at[pl.ds(0, 32)])
    pltpu.sync_copy(ghi_hbm, gb_v.at[pl.ds(32, 32)])
    lane16 = lax.iota(jnp.int32, 16)
    lmask = lane16 == (w & 15)
    ghalf = w < NUM_SUBCORES

    def pick(base):
        va = gb_v[pl.ds(base, 16)]
        vb = gb_v[pl.ds(base + 16, 16)]
        v = jnp.where(ghalf, va, vb)
        return jnp.max(jnp.where(lmask, v, jnp.int32(-1)))

    glo = pick(0)
    ghi = pick(32)
    start = pl.multiple_of(glo & jnp.int32(~7), 8)
    cnt = ghi - start
    nsteps = (cnt + (K - 1)) >> 7
    n4 = (nsteps + 3) >> 2

    row_lo = w * ROWS_PER_TILE
    row_lo_v = jnp.full((16,), row_lo, dtype=jnp.int32)

    def idx_refs(bi):
        return ((row_hbm, rvs[bi]), (colp_hbm, cvs[bi]),
                (w0_hbm, w0s[bi]), (w1_hbm, w1s[bi]))

    def issue_idx(t, bi):
        base = start + t * K
        for hbm, vm in idx_refs(bi):
            pltpu.async_copy(hbm.at[pl.ds(base, K)], vm, sem_i[bi])

    def wait_idx(t, bi):
        base = start + t * K
        for hbm, vm in idx_refs(bi):
            pltpu.make_async_copy(hbm.at[pl.ds(base, K)], vm,
                                  sem_i[bi]).wait()

    def issue_gather(bi, bg):
        pltpu.async_copy(emb_hbm.at[cvs[bi]], rows[bg], sem_g[bg])

    def wait_gather(bi, bg):
        pltpu.make_async_copy(emb_hbm.at[cvs[bi]], rows[bg],
                              sem_g[bg]).wait()

    @pl.when(nsteps >= 1)
    def _():
        issue_idx(0, 0)
        wait_idx(0, 0)
        issue_gather(0, 0)

    @pl.when(nsteps >= 2)
    def _():
        issue_idx(1, 1)

    qofs = tuple(lane16 + q * 16 for q in range(4))

    def bcast(vec, l):
        return vec.at[jnp.full((16,), l, dtype=jnp.int32)].get(
            mode="promise_in_bounds")

    def outer(t4, carry):
        for tb in range(4):
            t = 4 * t4 + tb

            @pl.when(t < nsteps)
            def _(tb=tb, t=t):
                wait_gather(tb, tb % 2)

                @pl.when(t + 1 < nsteps)
                def _():
                    wait_idx(t + 1, (tb + 1) % 4)
                    issue_gather((tb + 1) % 4, (tb + 1) % 2)

                def scale(j, acc_c, tb=tb):
                    sl16 = pl.ds(j * 16, 16)
                    rv = rvs[tb][sl16]
                    rl = rv - row_lo_v
                    inw = (rl >= 0) & (rl < ROWS_PER_TILE)
                    mf = inw.astype(jnp.float32)
                    rlc = jnp.clip(rl, 0, ROWS_PER_TILE - 1)
                    v0 = w0s[tb][sl16] * mf
                    v1 = w1s[tb][sl16] * mf
                    for l in range(16):
                        b0 = bcast(v0, l)
                        b1 = bcast(v1, l)
                        ib = bcast(rlc, l) << 6
                        i = j * 16 + l
                        for q in range(4):
                            a = rows[tb % 2][i, pl.ds(q * 16, 16)]
                            b = rows[tb % 2][i, pl.ds(64 + q * 16, 16)]
                            plsc.addupdate_scatter(acc, [ib + qofs[q]],
                                                   a * b0 + b * b1)
                    return acc_c

                lax.fori_loop(0, K // 16, scale, 0)

                @pl.when(t + 2 < nsteps)
                def _():
                    issue_idx(t + 2, (tb + 2) % 4)
        return carry

    lax.fori_loop(0, n4, outer, 0)

    pltpu.sync_copy(acc, out_hbm.at[pl.ds(pl.multiple_of(w * ACCW, 8),
                                          ACCW)])


_sc_call = pl.kernel(
    _sc_body,
    out_type=jax.ShapeDtypeStruct((NUM_IDS * EMBED_DIM,), jnp.float32),
    mesh=plsc.VectorSubcoreMesh(core_axis_name="c", subcore_axis_name="s",
                                num_cores=NUM_CORES,
                                num_subcores=NUM_SUBCORES),
    scratch_types=(
        [pltpu.VMEM((64,), jnp.int32)]
        + [pltpu.VMEM((K,), jnp.int32) for _ in range(2 * NBI)]
        + [pltpu.VMEM((K,), jnp.float32) for _ in range(2 * NBI)]
        + [pltpu.VMEM((K, 2 * EMBED_DIM), jnp.float32),
           pltpu.VMEM((K, 2 * EMBED_DIM), jnp.float32),
           pltpu.VMEM((ACCW,), jnp.float32)]
        + [pltpu.SemaphoreType.DMA for _ in range(6)]
    ),
    compiler_params=pltpu.CompilerParams(use_tc_tiling_on_sc=True,
                                         needs_layout_passes=False),
)


def kernel(row_idx, col_idx, data_tensor, num_ids, embeddings):
    del num_ids
    row_idx = row_idx.astype(jnp.int32)
    col_idx = col_idx.astype(jnp.int32)
    cpar = (col_idx & 1).astype(jnp.float32)
    w0 = data_tensor * (1.0 - cpar)
    w1 = data_tensor * cpar
    colp = col_idx >> 1
    rowpad = jnp.pad(row_idx, (0, PAD))
    colpad = jnp.pad(colp, (0, PAD))
    w0pad = jnp.pad(w0, (0, PAD))
    w1pad = jnp.pad(w1, (0, PAD))
    edges = jnp.arange(NUM_WORKERS + 1, dtype=jnp.int32) * ROWS_PER_TILE
    gb = jnp.searchsorted(row_idx, edges).astype(jnp.int32)
    glo = gb[:NUM_WORKERS]
    ghi = gb[1:]
    emb_pairs = embeddings.reshape(NTAB, 2 * EMBED_DIM)
    flat = _sc_call(rowpad, colpad, w0pad, w1pad, glo, ghi, emb_pairs)
    return flat.reshape(NUM_IDS, EMBED_DIM)


# speedup vs baseline: 1.1702x; 1.1702x over previous
import jax
import jax.numpy as jnp
from jax import lax
from jax.experimental import pallas as pl
from jax.experimental.pallas import tpu as pltpu
from jax.experimental.pallas import tpu_sc as plsc

NNZ = 819200
NUM_IDS = 16384
EMBED_DIM = 64
NUM_CORES = 2
NUM_SUBCORES = 16
NUM_WORKERS = NUM_CORES * NUM_SUBCORES
ROWS_PER_TILE = NUM_IDS // NUM_WORKERS
ACCW = ROWS_PER_TILE * EMBED_DIM
K = 128
PAD = 1024
NBI = 4


def _sc_body(row_hbm, col_hbm, wt_hbm, glo_hbm, ghi_hbm,
             emb_hbm, out_hbm,
             gb_v,
             r0, r1, r2, r3, c0, c1, c2, c3,
             wa0, wa1, wa2, wa3,
             rows0, rows1, acc,
             si0, si1, si2, si3, sg0, sg1):
    rvs = (r0, r1, r2, r3)
    cvs = (c0, c1, c2, c3)
    wts = (wa0, wa1, wa2, wa3)
    rows = (rows0, rows1)
    sem_i = (si0, si1, si2, si3)
    sem_g = (sg0, sg1)

    c = lax.axis_index("c")
    s = lax.axis_index("s")
    w = c * NUM_SUBCORES + s

    zv = jnp.zeros((16,), jnp.float32)

    def zacc(j, carry):
        acc[pl.ds(j * 16, 16)] = zv
        return carry

    lax.fori_loop(0, ACCW // 16, zacc, 0)

    pltpu.sync_copy(glo_hbm, gb_v.at[pl.ds(0, 32)])
    pltpu.sync_copy(ghi_hbm, gb_v.at[pl.ds(32, 32)])
    lane16 = lax.iota(jnp.int32, 16)
    lmask = lane16 == (w & 15)
    ghalf = w < NUM_SUBCORES

    def pick(base):
        va = gb_v[pl.ds(base, 16)]
        vb = gb_v[pl.ds(base + 16, 16)]
        v = jnp.where(ghalf, va, vb)
        return jnp.max(jnp.where(lmask, v, jnp.int32(-1)))

    glo = pick(0)
    ghi = pick(32)
    start = pl.multiple_of(glo & jnp.int32(~7), 8)
    cnt = ghi - start
    nsteps = (cnt + (K - 1)) >> 7
    n4 = (nsteps + 3) >> 2

    row_lo = w * ROWS_PER_TILE
    row_lo_v = jnp.full((16,), row_lo, dtype=jnp.int32)

    def idx_refs(bi):
        return ((row_hbm, rvs[bi]), (col_hbm, cvs[bi]), (wt_hbm, wts[bi]))

    def issue_idx(t, bi):
        base = start + t * K
        for hbm, vm in idx_refs(bi):
            pltpu.async_copy(hbm.at[pl.ds(base, K)], vm, sem_i[bi])

    def wait_idx(t, bi):
        base = start + t * K
        for hbm, vm in idx_refs(bi):
            pltpu.make_async_copy(hbm.at[pl.ds(base, K)], vm,
                                  sem_i[bi]).wait()

    def issue_gather(bi, bg):
        pltpu.async_copy(emb_hbm.at[cvs[bi]], rows[bg], sem_g[bg])

    def wait_gather(bi, bg):
        pltpu.make_async_copy(emb_hbm.at[cvs[bi]], rows[bg],
                              sem_g[bg]).wait()

    @pl.when(nsteps >= 1)
    def _():
        issue_idx(0, 0)
        wait_idx(0, 0)
        issue_gather(0, 0)

    @pl.when(nsteps >= 2)
    def _():
        issue_idx(1, 1)

    qofs = tuple(lane16 + q * 16 for q in range(4))

    def bcast(vec, l):
        return vec.at[jnp.full((16,), l, dtype=jnp.int32)].get(
            mode="promise_in_bounds")

    def outer(t4, carry):
        for tb in range(4):
            t = 4 * t4 + tb

            @pl.when(t < nsteps)
            def _(tb=tb, t=t):
                wait_gather(tb, tb % 2)

                @pl.when(t + 1 < nsteps)
                def _():
                    wait_idx(t + 1, (tb + 1) % 4)
                    issue_gather((tb + 1) % 4, (tb + 1) % 2)

                def scale(j, acc_c, tb=tb):
                    sl16 = pl.ds(j * 16, 16)
                    rv = rvs[tb][sl16]
                    rl = rv - row_lo_v
                    inw = (rl >= 0) & (rl < ROWS_PER_TILE)
                    mf = inw.astype(jnp.float32)
                    rlc = jnp.clip(rl, 0, ROWS_PER_TILE - 1)
                    v0 = wts[tb][sl16] * mf
                    for l in range(16):
                        b0 = bcast(v0, l)
                        ib = bcast(rlc, l) << 6
                        i = j * 16 + l
                        for q in range(4):
                            a = rows[tb % 2][i, pl.ds(q * 16, 16)]
                            plsc.addupdate_scatter(acc, [ib + qofs[q]],
                                                   a * b0)
                    return acc_c

                lax.fori_loop(0, K // 16, scale, 0)

                @pl.when(t + 2 < nsteps)
                def _():
                    issue_idx(t + 2, (tb + 2) % 4)
        return carry

    lax.fori_loop(0, n4, outer, 0)

    pltpu.sync_copy(acc, out_hbm.at[pl.ds(pl.multiple_of(w * ACCW, 8),
                                          ACCW)])


_sc_call = pl.kernel(
    _sc_body,
    out_type=jax.ShapeDtypeStruct((NUM_IDS * EMBED_DIM,), jnp.float32),
    mesh=plsc.VectorSubcoreMesh(core_axis_name="c", subcore_axis_name="s",
                                num_cores=NUM_CORES,
                                num_subcores=NUM_SUBCORES),
    scratch_types=(
        [pltpu.VMEM((64,), jnp.int32)]
        + [pltpu.VMEM((K,), jnp.int32) for _ in range(2 * NBI)]
        + [pltpu.VMEM((K,), jnp.float32) for _ in range(NBI)]
        + [pltpu.VMEM((K, EMBED_DIM), jnp.float32),
           pltpu.VMEM((K, EMBED_DIM), jnp.float32),
           pltpu.VMEM((ACCW,), jnp.float32)]
        + [pltpu.SemaphoreType.DMA for _ in range(6)]
    ),
    compiler_params=pltpu.CompilerParams(use_tc_tiling_on_sc=False,
                                         needs_layout_passes=False),
)


def kernel(row_idx, col_idx, data_tensor, num_ids, embeddings):
    del num_ids
    row_idx = row_idx.astype(jnp.int32)
    col_idx = col_idx.astype(jnp.int32)
    rowpad = jnp.pad(row_idx, (0, PAD))
    colpad = jnp.pad(col_idx, (0, PAD))
    wpad = jnp.pad(data_tensor, (0, PAD))
    edges = jnp.arange(NUM_WORKERS + 1, dtype=jnp.int32) * ROWS_PER_TILE
    gb = jnp.searchsorted(row_idx, edges).astype(jnp.int32)
    glo = gb[:NUM_WORKERS]
    ghi = gb[1:]
    flat = _sc_call(rowpad, colpad, wpad, glo, ghi, embeddings)
    return flat.reshape(NUM_IDS, EMBED_DIM)
